# Initial kernel scaffold; baseline (speedup 1.0000x reference)
import jax
import jax.numpy as jnp
from jax.experimental import pallas as pl


def kernel(indices, weight_table):
    raise NotImplementedError("write your pallas kernel here")


# speedup vs baseline: 1.0848x; 1.0848x over previous
import functools

import jax
import jax.numpy as jnp
from jax import lax
from jax.experimental import pallas as pl
from jax.experimental.pallas import tpu as pltpu
from jax.experimental.pallas import tpu_sc as plsc

BATCH = 16384
A_MIN = 0.0
A_MAX = 2.0

LANES = 16
CHUNK = 128
NUM_WORKERS = 32
ROWS_PER_W = BATCH // (NUM_WORKERS * CHUNK)

_mesh = plsc.VectorSubcoreMesh(core_axis_name="c", subcore_axis_name="s")


@functools.partial(
    pl.kernel,
    out_type=jax.ShapeDtypeStruct((NUM_WORKERS * ROWS_PER_W, CHUNK), jnp.float32),
    mesh=_mesh,
    scratch_types=[
        pltpu.VMEM((ROWS_PER_W, CHUNK), jnp.int32),
        pltpu.VMEM((ROWS_PER_W, CHUNK), jnp.float32),
        pltpu.SemaphoreType.DMA,
    ],
)
def _gather_clamp(idx_hbm, table_hbm, out_hbm, idx_v, rows_v, sem):
    wid = lax.axis_index("s") * 2 + lax.axis_index("c")
    row0 = wid * ROWS_PER_W

    pltpu.sync_copy(idx_hbm.at[pl.ds(row0, ROWS_PER_W)], idx_v)

    copies = [
        pltpu.async_copy(table_hbm.at[idx_v.at[j]], rows_v.at[j], sem)
        for j in range(ROWS_PER_W)
    ]
    for c in copies:
        c.wait()

    for j in range(ROWS_PER_W):
        for i in range(CHUNK // LANES):
            sl = pl.ds(i * LANES, LANES)
            v = rows_v[j, sl]
            rows_v[j, sl] = jnp.minimum(jnp.maximum(v, A_MIN), A_MAX)

    pltpu.sync_copy(rows_v, out_hbm.at[pl.ds(row0, ROWS_PER_W)])


def kernel(indices, weight_table):
    idx2 = indices.astype(jnp.int32).reshape(NUM_WORKERS * ROWS_PER_W, CHUNK)
    out = _gather_clamp(idx2, weight_table)
    return out.reshape(BATCH)


# speedup vs baseline: 1.0919x; 1.0066x over previous
import functools

import jax
import jax.numpy as jnp
from jax import lax
from jax.experimental import pallas as pl
from jax.experimental.pallas import tpu as pltpu
from jax.experimental.pallas import tpu_sc as plsc

BATCH = 16384
A_MIN = 0.0
A_MAX = 2.0

LANES = 16
CHUNK = 128
NUM_WORKERS = 32
ROWS_PER_W = BATCH // (NUM_WORKERS * CHUNK)

_mesh = plsc.VectorSubcoreMesh(core_axis_name="c", subcore_axis_name="s")


@functools.partial(
    pl.kernel,
    out_type=jax.ShapeDtypeStruct((BATCH,), jnp.float32),
    mesh=_mesh,
    scratch_types=[
        pltpu.VMEM((ROWS_PER_W * CHUNK,), jnp.int32),
        pltpu.VMEM((ROWS_PER_W * CHUNK,), jnp.float32),
        pltpu.SemaphoreType.DMA,
    ],
)
def _gather_clamp(idx_hbm, table_hbm, out_hbm, idx_v, rows_v, sem):
    wid = lax.axis_index("s") * 2 + lax.axis_index("c")
    base = wid * ROWS_PER_W * CHUNK

    pltpu.sync_copy(idx_hbm.at[pl.ds(base, ROWS_PER_W * CHUNK)], idx_v)

    pltpu.async_copy(table_hbm.at[idx_v], rows_v, sem).wait()

    for i in range(ROWS_PER_W * CHUNK // LANES):
        sl = pl.ds(i * LANES, LANES)
        v = rows_v[sl]
        rows_v[sl] = jnp.minimum(jnp.maximum(v, A_MIN), A_MAX)

    pltpu.sync_copy(rows_v, out_hbm.at[pl.ds(base, ROWS_PER_W * CHUNK)])


def kernel(indices, weight_table):
    idx = indices.astype(jnp.int32)
    return _gather_clamp(idx, weight_table)


# speedup vs baseline: 1.0982x; 1.0057x over previous
import functools

import jax
import jax.numpy as jnp
from jax import lax
from jax.experimental import pallas as pl
from jax.experimental.pallas import tpu as pltpu
from jax.experimental.pallas import tpu_sc as plsc

BATCH = 16384
A_MIN = 0.0
A_MAX = 2.0

LANES = 16
CHUNK = 128
NUM_WORKERS = 32
ROWS_PER_W = BATCH // (NUM_WORKERS * CHUNK)

_mesh = plsc.VectorSubcoreMesh(core_axis_name="c", subcore_axis_name="s")


@functools.partial(
    pl.kernel,
    out_type=jax.ShapeDtypeStruct((BATCH,), jnp.float32),
    mesh=_mesh,
    scratch_types=[
        pltpu.VMEM((ROWS_PER_W * CHUNK,), jnp.int32),
        pltpu.VMEM((ROWS_PER_W * CHUNK,), jnp.float32),
        [pltpu.SemaphoreType.DMA] * ROWS_PER_W,
        pltpu.SemaphoreType.DMA,
    ],
)
def _gather_clamp(idx_hbm, table_hbm, out_hbm, idx_v, rows_v, gsems, wsem):
    wid = lax.axis_index("s") * 2 + lax.axis_index("c")
    base = wid * ROWS_PER_W * CHUNK

    pltpu.sync_copy(idx_hbm.at[pl.ds(base, ROWS_PER_W * CHUNK)], idx_v)

    gathers = [
        pltpu.async_copy(
            table_hbm.at[idx_v.at[pl.ds(j * CHUNK, CHUNK)]],
            rows_v.at[pl.ds(j * CHUNK, CHUNK)],
            gsems[j],
        )
        for j in range(ROWS_PER_W)
    ]

    writes = []
    for j in range(ROWS_PER_W):
        gathers[j].wait()
        for i in range(CHUNK // LANES):
            sl = pl.ds(j * CHUNK + i * LANES, LANES)
            v = rows_v[sl]
            rows_v[sl] = jnp.minimum(jnp.maximum(v, A_MIN), A_MAX)
        writes.append(
            pltpu.async_copy(
                rows_v.at[pl.ds(j * CHUNK, CHUNK)],
                out_hbm.at[pl.ds(base + j * CHUNK, CHUNK)],
                wsem,
            )
        )
    for w in writes:
        w.wait()


def kernel(indices, weight_table):
    idx = indices.astype(jnp.int32)
    return _gather_clamp(idx, weight_table)


# speedup vs baseline: 1.1138x; 1.0142x over previous
import functools

import jax
import jax.numpy as jnp
from jax import lax
from jax.experimental import pallas as pl
from jax.experimental.pallas import tpu as pltpu
from jax.experimental.pallas import tpu_sc as plsc

BATCH = 16384
A_MIN = 0.0
A_MAX = 2.0

LANES = 16
CHUNK = 256
NUM_WORKERS = 32
ROWS_PER_W = BATCH // (NUM_WORKERS * CHUNK)

_mesh = plsc.VectorSubcoreMesh(core_axis_name="c", subcore_axis_name="s")


@functools.partial(
    pl.kernel,
    out_type=jax.ShapeDtypeStruct((BATCH,), jnp.float32),
    mesh=_mesh,
    scratch_types=[
        pltpu.VMEM((ROWS_PER_W * CHUNK,), jnp.int32),
        pltpu.VMEM((ROWS_PER_W * CHUNK,), jnp.float32),
        [pltpu.SemaphoreType.DMA] * ROWS_PER_W,
        pltpu.SemaphoreType.DMA,
    ],
)
def _gather_clamp(idx_hbm, table_hbm, out_hbm, idx_v, rows_v, gsems, wsem):
    wid = lax.axis_index("s") * 2 + lax.axis_index("c")
    base = wid * ROWS_PER_W * CHUNK

    pltpu.sync_copy(idx_hbm.at[pl.ds(base, ROWS_PER_W * CHUNK)], idx_v)

    gathers = [
        pltpu.async_copy(
            table_hbm.at[idx_v.at[pl.ds(j * CHUNK, CHUNK)]],
            rows_v.at[pl.ds(j * CHUNK, CHUNK)],
            gsems[j],
        )
        for j in range(ROWS_PER_W)
    ]

    writes = []
    for j in range(ROWS_PER_W):
        gathers[j].wait()
        for i in range(CHUNK // LANES):
            sl = pl.ds(j * CHUNK + i * LANES, LANES)
            v = rows_v[sl]
            rows_v[sl] = jnp.minimum(jnp.maximum(v, A_MIN), A_MAX)
        writes.append(
            pltpu.async_copy(
                rows_v.at[pl.ds(j * CHUNK, CHUNK)],
                out_hbm.at[pl.ds(base + j * CHUNK, CHUNK)],
                wsem,
            )
        )
    for w in writes:
        w.wait()


def kernel(indices, weight_table):
    idx = indices.astype(jnp.int32)
    return _gather_clamp(idx, weight_table)
